# baseline (speedup 1.0000x reference)
import functools

import jax
import jax.numpy as jnp
from jax import lax
from jax.experimental import pallas as pl
from jax.experimental.pallas import tpu as pltpu
from jax.experimental.pallas import tpu_sc as plsc

NC, NS = 2, 16
NW = NC * NS
CHUNK = 32


def _sc_body(B, S, D, x_hbm, tid_hbm, p_hbm, o_hbm,
             buf0, buf1, idx_v, prow_v, insem, outsem, psem):
    wid = lax.axis_index("s") * NC + lax.axis_index("c")
    per_b = NW // B
    rows_w = S // per_b
    b = wid // per_b
    base = (wid % per_b) * rows_w
    n_chunks = rows_w // CHUNK
    bufs = (buf0, buf1)

    @pl.when(wid == 0)
    def _lookup():
        idx_cp = pltpu.make_async_copy(tid_hbm, idx_v, psem)
        idx_cp.start()
        idx_cp.wait()
        gather = pltpu.make_async_copy(p_hbm.at[idx_v], prow_v, psem)
        gather.start()
        gather.wait()
        for bb in range(B):
            pltpu.sync_copy(prow_v.at[pl.ds(bb, 1), :],
                            o_hbm.at[bb, pl.ds(S, 1), :])

    def in_cp(k):
        return pltpu.make_async_copy(
            x_hbm.at[b, pl.ds(base + k * CHUNK, CHUNK), :],
            bufs[k % 2], insem.at[k % 2])

    def out_cp(k):
        return pltpu.make_async_copy(
            bufs[k % 2],
            o_hbm.at[b, pl.ds(base + k * CHUNK, CHUNK), :],
            outsem.at[k % 2])

    in_cp(0).start()
    if n_chunks > 1:
        in_cp(1).start()
    for k in range(n_chunks):
        in_cp(k).wait()
        out_cp(k).start()
        if k + 2 < n_chunks:
            out_cp(k).wait()
            in_cp(k + 2).start()
    if n_chunks > 1:
        out_cp(n_chunks - 2).wait()
    out_cp(n_chunks - 1).wait()


def kernel(x, task_id, prompt):
    B, S, D = x.shape

    body = functools.partial(_sc_body, B, S, D)
    run = pl.kernel(
        body,
        out_type=jax.ShapeDtypeStruct((B, S + 1, D), x.dtype),
        mesh=plsc.VectorSubcoreMesh(
            core_axis_name="c", subcore_axis_name="s",
            num_cores=NC, num_subcores=NS),
        scratch_types=[
            pltpu.VMEM((CHUNK, D), jnp.float32),
            pltpu.VMEM((CHUNK, D), jnp.float32),
            pltpu.VMEM((B,), jnp.int32),
            pltpu.VMEM((B, D), jnp.float32),
            pltpu.SemaphoreType.DMA((2,)),
            pltpu.SemaphoreType.DMA((2,)),
            pltpu.SemaphoreType.DMA,
        ],
    )
    out = run(x, task_id, prompt)
    return (out, task_id)


# speedup vs baseline: 2.9361x; 2.9361x over previous
import jax
import jax.numpy as jnp
from jax.experimental import pallas as pl
from jax.experimental.pallas import tpu as pltpu

SEQ_BLOCK = 256


def _body(t_ref, x_ref, p_ref, o_ref, pscr_ref):
    s = pl.program_id(0)
    ns = pl.num_programs(0)
    B = x_ref.shape[0]

    @pl.when(s < B)
    def _stash_prompt_row():
        pscr_ref[s, :] = p_ref[0, 0, :]

    @pl.when(s < ns - 1)
    def _copy():
        for b in range(B):
            o_ref[:, b, :] = x_ref[b, :, :]

    @pl.when(s == ns - 1)
    def _prompt_rows():
        o_ref[0, :, :] = pscr_ref[...]


def kernel(x, task_id, prompt):
    B, S, D = x.shape
    n_sb = S // SEQ_BLOCK
    prompt3 = prompt.reshape(prompt.shape[0], 1, D)

    grid_spec = pltpu.PrefetchScalarGridSpec(
        num_scalar_prefetch=1,
        grid=(n_sb + 1,),
        in_specs=[
            pl.BlockSpec((B, SEQ_BLOCK, D),
                         lambda s, t: (0, jnp.minimum(s, n_sb - 1), 0)),
            pl.BlockSpec((1, 1, D),
                         lambda s, t: (t[jnp.minimum(s, B - 1)], 0, 0)),
        ],
        out_specs=pl.BlockSpec((SEQ_BLOCK, B, D), lambda s, t: (s, 0, 0)),
        scratch_shapes=[pltpu.VMEM((B, D), jnp.float32)],
    )
    out_t = pl.pallas_call(
        _body,
        grid_spec=grid_spec,
        out_shape=jax.ShapeDtypeStruct((S + 1, B, D), x.dtype),
    )(task_id, x, prompt3)
    out = jnp.transpose(out_t, (1, 0, 2))
    return (out, task_id)


# speedup vs baseline: 3.6865x; 1.2556x over previous
import jax
import jax.numpy as jnp
from jax.experimental import pallas as pl
from jax.experimental.pallas import tpu as pltpu

SEQ_BLOCK = 256


def _body(t_ref, x_ref, p_ref, o_ref, pscr_ref, psem):
    s = pl.program_id(0)
    ns = pl.num_programs(0)
    B = x_ref.shape[0]

    def row_cp(b):
        return pltpu.make_async_copy(
            p_ref.at[pl.ds(t_ref[b], 1), :],
            pscr_ref.at[pl.ds(b, 1), :],
            psem)

    @pl.when(s == 0)
    def _start_lookup():
        for b in range(B):
            row_cp(b).start()

    @pl.when(s < ns - 1)
    def _copy():
        for b in range(B):
            o_ref[:, b, :] = x_ref[b, :, :]

    @pl.when(s == ns - 1)
    def _prompt_rows():
        for b in range(B):
            row_cp(b).wait()
        o_ref[0, :, :] = pscr_ref[...]


def kernel(x, task_id, prompt):
    B, S, D = x.shape
    n_sb = S // SEQ_BLOCK

    grid_spec = pltpu.PrefetchScalarGridSpec(
        num_scalar_prefetch=1,
        grid=(n_sb + 1,),
        in_specs=[
            pl.BlockSpec((B, SEQ_BLOCK, D),
                         lambda s, t: (0, jnp.minimum(s, n_sb - 1), 0)),
            pl.BlockSpec(memory_space=pl.ANY),
        ],
        out_specs=pl.BlockSpec((SEQ_BLOCK, B, D), lambda s, t: (s, 0, 0)),
        scratch_shapes=[
            pltpu.VMEM((B, D), jnp.float32),
            pltpu.SemaphoreType.DMA,
        ],
    )
    out_t = pl.pallas_call(
        _body,
        grid_spec=grid_spec,
        out_shape=jax.ShapeDtypeStruct((S + 1, B, D), x.dtype),
    )(task_id, x, prompt)
    out = jnp.transpose(out_t, (1, 0, 2))
    return (out, task_id)


# speedup vs baseline: 3.7956x; 1.0296x over previous
import jax
import jax.numpy as jnp
from jax.experimental import pallas as pl
from jax.experimental.pallas import tpu as pltpu

SEQ_BLOCK = 512


def _body(t_ref, x_ref, p_ref, o_ref, pscr_ref, psem):
    s = pl.program_id(0)
    ns = pl.num_programs(0)
    B = x_ref.shape[0]

    def row_cp(b):
        return pltpu.make_async_copy(
            p_ref.at[pl.ds(t_ref[b], 1), :],
            pscr_ref.at[pl.ds(b, 1), :],
            psem)

    @pl.when(s == 0)
    def _start_lookup():
        for b in range(B):
            row_cp(b).start()

    @pl.when(s < ns - 1)
    def _copy():
        for b in range(B):
            o_ref[:, b, :] = x_ref[b, :, :]

    @pl.when(s == ns - 1)
    def _prompt_rows():
        for b in range(B):
            row_cp(b).wait()
        o_ref[0, :, :] = pscr_ref[...]


def kernel(x, task_id, prompt):
    B, S, D = x.shape
    n_sb = S // SEQ_BLOCK

    grid_spec = pltpu.PrefetchScalarGridSpec(
        num_scalar_prefetch=1,
        grid=(n_sb + 1,),
        in_specs=[
            pl.BlockSpec((B, SEQ_BLOCK, D),
                         lambda s, t: (0, jnp.minimum(s, n_sb - 1), 0)),
            pl.BlockSpec(memory_space=pl.ANY),
        ],
        out_specs=pl.BlockSpec((SEQ_BLOCK, B, D), lambda s, t: (s, 0, 0)),
        scratch_shapes=[
            pltpu.VMEM((B, D), jnp.float32),
            pltpu.SemaphoreType.DMA,
        ],
    )
    out_t = pl.pallas_call(
        _body,
        grid_spec=grid_spec,
        out_shape=jax.ShapeDtypeStruct((S + 1, B, D), x.dtype),
    )(task_id, x, prompt)
    out = jnp.transpose(out_t, (1, 0, 2))
    return (out, task_id)
